# baseline (speedup 1.0000x reference)
import functools

import jax
import jax.numpy as jnp
from jax import lax
from jax.experimental import pallas as pl
from jax.experimental.pallas import tpu as pltpu
from jax.experimental.pallas import tpu_sc as plsc

_B = 16384
_D = 64
_A = 18
_N = 1_000_000
_NQ = 14
_NPACK = 73728
_NC = 2
_NS = 16
_NW = _NC * _NS
_BPW = _B // _NW
_L = 16

_BN = 6144
_GRID = _NPACK // _BN
_LAST_BLK = _N // _BN


def _proj_body(*refs):
    t_refs = refs[:_NQ]
    we_ref, wo_ref, be_ref, bo_ref, o_ref = refs[_NQ:]
    g = pl.program_id(0)
    col = jax.lax.broadcasted_iota(jnp.int32, (1, _BN), 1)
    slabs = [t_ref[...].astype(jnp.bfloat16) for t_ref in t_refs[:-1]]
    start = jnp.minimum((_NQ - 1) * _GRID + g, _LAST_BLK) * _BN
    slabs.append(
        jnp.where(start + col < _N, t_refs[-1][...], 0.0).astype(jnp.bfloat16)
    )
    lhs = jnp.concatenate(slabs, axis=0)
    de = lax.dot_general(
        lhs, we_ref[...], (((0,), (0,)), ((), ())),
        preferred_element_type=jnp.float32,
    ) + be_ref[...]
    do = lax.dot_general(
        lhs, wo_ref[...], (((0,), (0,)), ((), ())),
        preferred_element_type=jnp.float32,
    ) + bo_ref[...]
    eu = lax.bitcast_convert_type(de.astype(jnp.bfloat16), jnp.uint16)
    ou = lax.bitcast_convert_type(do.astype(jnp.bfloat16), jnp.uint16)
    word = (ou.astype(jnp.uint32) << 16) | eu.astype(jnp.uint32)
    o_ref[...] = lax.bitcast_convert_type(word, jnp.int32)


@functools.cache
def _make_gather_sc():
    @functools.partial(
        pl.kernel,
        out_type=jax.ShapeDtypeStruct((_A, _B), jnp.float32),
        mesh=plsc.VectorSubcoreMesh(core_axis_name="c", subcore_axis_name="s"),
        scratch_types=[
            pltpu.VMEM((3, _BPW), jnp.int32),
            pltpu.VMEM((4, 128), jnp.int32),
            pltpu.VMEM((4, 128), jnp.int32),
            pltpu.VMEM((_BPW, 128), jnp.int32),
            pltpu.VMEM((_A, _BPW), jnp.float32),
            pltpu.SemaphoreType.DMA,
            pltpu.SemaphoreType.DMA,
        ],
        compiler_params=pltpu.CompilerParams(needs_layout_passes=False),
    )
    def _k(state_hbm, tp_hbm, out_hbm, sv, idx_v, off_v, rows_v, dest_v, sem,
           osem):
        wid = lax.axis_index("s") * _NC + lax.axis_index("c")
        base = wid * _BPW
        pltpu.sync_copy(state_hbm.at[:, pl.ds(base, _BPW)], sv)
        copies = []
        for j in range(4):
            for i in range(8):
                sl = pl.ds(j * 128 + i * _L, _L)
                ids = sv[0, sl] * 10000 + sv[1, sl] * 100 + sv[2, sl]
                q = (ids >= _NPACK).astype(jnp.int32)
                for k in range(2, _NQ):
                    q = q + (ids >= k * _NPACK).astype(jnp.int32)
                idx_v[j, pl.ds(i * _L, _L)] = ids - q * _NPACK
                off_v[j, pl.ds(i * _L, _L)] = q * (_A // 2)
            copies.append(pltpu.async_copy(
                tp_hbm.at[idx_v.at[j]], rows_v.at[pl.ds(j * 128, 128)], sem
            ))
        lane = jax.lax.iota(jnp.int32, _L)
        for j in range(4):
            copies[j].wait()
            for gi in range(8):
                g = j * 8 + gi
                rows16 = lane + g * _L
                offs = off_v[j, pl.ds(gi * _L, _L)]
                for a in range(_A):
                    w = plsc.load_gather(rows_v, [rows16, offs + (a // 2)])
                    if a % 2 == 0:
                        hi = w << 16
                    else:
                        hi = w & jnp.int32(-65536)
                    dest_v[a, pl.ds(g * _L, _L)] = plsc.bitcast(
                        hi, jnp.float32
                    )
            pltpu.async_copy(
                dest_v.at[:, pl.ds(j * 128, 128)],
                out_hbm.at[:, pl.ds(base + j * 128, 128)],
                osem,
            )
        pltpu.make_async_copy(
            out_hbm.at[:, pl.ds(0, _BPW)], dest_v, osem
        ).wait()

    return _k


def _pack_weights(W, b):
    we = jnp.zeros((_NQ * _D, 128), jnp.float32)
    wo = jnp.zeros((_NQ * _D, 128), jnp.float32)
    be = jnp.zeros((1, 128), jnp.float32)
    bo = jnp.zeros((1, 128), jnp.float32)
    for q in range(_NQ):
        we = we.at[q * _D:(q + 1) * _D, q * 9:q * 9 + 9].set(W[:, 0::2])
        wo = wo.at[q * _D:(q + 1) * _D, q * 9:q * 9 + 9].set(W[:, 1::2])
        be = be.at[0, q * 9:q * 9 + 9].set(b[0::2])
        bo = bo.at[0, q * 9:q * 9 + 9].set(b[1::2])
    return we.astype(jnp.bfloat16), wo.astype(jnp.bfloat16), be, bo


def kernel(state, embed_table, W, b):
    state_t = state.astype(jnp.int32).T
    table_t = embed_table.T
    we, wo, be, bo = _pack_weights(W, b)
    tp = pl.pallas_call(
        _proj_body,
        grid=(_GRID,),
        in_specs=[
            pl.BlockSpec(
                (_D, _BN),
                (lambda g, q=q: (0, jnp.minimum(q * _GRID + g, _LAST_BLK)))
                if q == _NQ - 1 else (lambda g, q=q: (0, q * _GRID + g)),
            )
            for q in range(_NQ)
        ] + [
            pl.BlockSpec((_NQ * _D, 128), lambda g: (0, 0)),
            pl.BlockSpec((_NQ * _D, 128), lambda g: (0, 0)),
            pl.BlockSpec((1, 128), lambda g: (0, 0)),
            pl.BlockSpec((1, 128), lambda g: (0, 0)),
        ],
        out_specs=pl.BlockSpec((_BN, 128), lambda g: (g, 0)),
        out_shape=jax.ShapeDtypeStruct((_NPACK, 128), jnp.int32),
        compiler_params=pltpu.CompilerParams(
            vmem_limit_bytes=100 * 1024 * 1024
        ),
    )(*([table_t] * _NQ), we, wo, be, bo)
    out_t = _make_gather_sc()(state_t, tp)
    return out_t.T


# speedup vs baseline: 1.7398x; 1.7398x over previous
import functools

import jax
import jax.numpy as jnp
from jax import lax
from jax.experimental import pallas as pl
from jax.experimental.pallas import tpu as pltpu
from jax.experimental.pallas import tpu_sc as plsc

_B = 16384
_D = 64
_A = 18
_N = 1_000_000
_NQ = 7
_NPACK = 147456
_NC = 2
_NS = 16
_NW = _NC * _NS
_BPW = _B // _NW
_L = 16

_BN = 12288
_GRID = _NPACK // _BN
_LAST_BLK = _N // _BN


def _proj_body(*refs):
    t_refs = refs[:_NQ]
    w7_ref, b128_ref, o_ref = refs[_NQ:]
    g = pl.program_id(0)
    col = jax.lax.broadcasted_iota(jnp.int32, (1, _BN), 1)
    slabs = [t_ref[...] for t_ref in t_refs[:-1]]
    start = jnp.minimum((_NQ - 1) * _GRID + g, _LAST_BLK) * _BN
    slabs.append(jnp.where(start + col < _N, t_refs[-1][...], 0.0))
    lhs = jnp.concatenate(slabs, axis=0)
    o_ref[...] = lax.dot_general(
        lhs, w7_ref[...], (((0,), (0,)), ((), ())),
        preferred_element_type=jnp.float32,
    ) + b128_ref[...]


@functools.cache
def _make_gather_sc():
    @functools.partial(
        pl.kernel,
        out_type=jax.ShapeDtypeStruct((_A, _B), jnp.float32),
        mesh=plsc.VectorSubcoreMesh(core_axis_name="c", subcore_axis_name="s"),
        scratch_types=[
            pltpu.VMEM((3, _BPW), jnp.int32),
            pltpu.VMEM((4, 128), jnp.int32),
            pltpu.VMEM((4, 128), jnp.int32),
            pltpu.VMEM((_BPW, 128), jnp.float32),
            pltpu.VMEM((_A, _BPW), jnp.float32),
            pltpu.SemaphoreType.DMA,
            pltpu.SemaphoreType.DMA,
        ],
        compiler_params=pltpu.CompilerParams(needs_layout_passes=False),
    )
    def _k(state_hbm, tp_hbm, out_hbm, sv, idx_v, off_v, rows_v, dest_v, sem,
           osem):
        wid = lax.axis_index("s") * _NC + lax.axis_index("c")
        base = wid * _BPW
        pltpu.sync_copy(state_hbm.at[:, pl.ds(base, _BPW)], sv)
        copies = []
        for j in range(4):
            for i in range(8):
                sl = pl.ds(j * 128 + i * _L, _L)
                ids = sv[0, sl] * 10000 + sv[1, sl] * 100 + sv[2, sl]
                q = (ids >= _NPACK).astype(jnp.int32)
                for k in range(2, _NQ):
                    q = q + (ids >= k * _NPACK).astype(jnp.int32)
                idx_v[j, pl.ds(i * _L, _L)] = ids - q * _NPACK
                off_v[j, pl.ds(i * _L, _L)] = q * _A
            copies.append(pltpu.async_copy(
                tp_hbm.at[idx_v.at[j]], rows_v.at[pl.ds(j * 128, 128)], sem
            ))
        lane = jax.lax.iota(jnp.int32, _L)
        for j in range(4):
            copies[j].wait()
            for gi in range(8):
                g = j * 8 + gi
                rows16 = lane + g * _L
                offs = off_v[j, pl.ds(gi * _L, _L)]
                for a in range(_A):
                    vals = plsc.load_gather(rows_v, [rows16, offs + a])
                    dest_v[a, pl.ds(g * _L, _L)] = vals
            pltpu.async_copy(
                dest_v.at[:, pl.ds(j * 128, 128)],
                out_hbm.at[:, pl.ds(base + j * 128, 128)],
                osem,
            )
        pltpu.make_async_copy(
            out_hbm.at[:, pl.ds(0, _BPW)], dest_v, osem
        ).wait()

    return _k


def kernel(state, embed_table, W, b):
    state_t = state.astype(jnp.int32).T
    table_t = embed_table.T
    w7 = jnp.zeros((_NQ * _D, 128), W.dtype)
    for q in range(_NQ):
        w7 = w7.at[q * _D:(q + 1) * _D, q * _A:(q + 1) * _A].set(W)
    b128 = jnp.concatenate([jnp.tile(b, _NQ), jnp.zeros((2,), b.dtype)])[None, :]
    tp = pl.pallas_call(
        _proj_body,
        grid=(_GRID,),
        in_specs=[
            pl.BlockSpec(
                (_D, _BN),
                (lambda g, q=q: (0, jnp.minimum(q * _GRID + g, _LAST_BLK)))
                if q == _NQ - 1 else (lambda g, q=q: (0, q * _GRID + g)),
            )
            for q in range(_NQ)
        ] + [
            pl.BlockSpec((_NQ * _D, 128), lambda g: (0, 0)),
            pl.BlockSpec((1, 128), lambda g: (0, 0)),
        ],
        out_specs=pl.BlockSpec((_BN, 128), lambda g: (g, 0)),
        out_shape=jax.ShapeDtypeStruct((_NPACK, 128), jnp.float32),
        compiler_params=pltpu.CompilerParams(
            vmem_limit_bytes=100 * 1024 * 1024
        ),
    )(*([table_t] * _NQ), w7, b128)
    out_t = _make_gather_sc()(state_t, tp)
    return out_t.T
